# baseline (speedup 1.0000x reference)
import functools

import jax
import jax.numpy as jnp
from jax import lax
from jax.experimental import pallas as pl
from jax.experimental.pallas import tpu as pltpu
from jax.experimental.pallas import tpu_sc as plsc

_NUM_CORES = 2
_NUM_SUBCORES = 16
_NUM_WORKERS = _NUM_CORES * _NUM_SUBCORES


@functools.partial(jax.jit, static_argnames=("n", "d"))
def _sc_gather(ids_flat, table, *, n, d):
    bpw = n // _NUM_WORKERS
    chunk = 16
    nchunk = bpw // chunk
    nbuf = 3
    mesh = plsc.VectorSubcoreMesh(core_axis_name="c", subcore_axis_name="s")

    @functools.partial(
        pl.kernel,
        mesh=mesh,
        out_type=jax.ShapeDtypeStruct((n, d), table.dtype),
        scratch_types=[
            pltpu.VMEM((bpw,), jnp.int32),
        ] + [pltpu.VMEM((chunk, d), table.dtype)] * nbuf
          + [pltpu.SemaphoreType.DMA] * (2 * nbuf),
    )
    def run(ids_hbm, table_hbm, out_hbm, idx_v, *bufs_and_sems):
        bufs = bufs_and_sems[:nbuf]
        sins = bufs_and_sems[nbuf:2 * nbuf]
        souts = bufs_and_sems[2 * nbuf:3 * nbuf]
        wid = lax.axis_index("s") * _NUM_CORES + lax.axis_index("c")
        base = wid * bpw
        pltpu.sync_copy(ids_hbm.at[pl.ds(base, bpw)], idx_v)

        def gather(g):
            k = g % nbuf
            return pltpu.make_async_copy(
                table_hbm.at[idx_v.at[pl.ds(g * chunk, chunk)]],
                bufs[k], sins[k])

        def put(g):
            k = g % nbuf
            return pltpu.make_async_copy(
                bufs[k], out_hbm.at[pl.ds(base + g * chunk, chunk)],
                souts[k])

        gathers, puts = {}, {}
        for g in range(min(2, nchunk)):
            gathers[g] = gather(g)
            gathers[g].start()
        for g in range(nchunk):
            if g + 1 < nchunk and g + 1 >= 2:
                if g - 2 >= 0:
                    puts[g - 2].wait()
                gathers[g + 1] = gather(g + 1)
                gathers[g + 1].start()
            gathers[g].wait()
            puts[g] = put(g)
            puts[g].start()
        for g in range(max(0, nchunk - 3), nchunk):
            puts[g].wait()

    return run(ids_flat, table)


def kernel(input_ids, embed_table):
    n = input_ids.size
    d = embed_table.shape[1]
    ids_flat = input_ids.reshape(-1).astype(jnp.int32)
    out = _sc_gather(ids_flat, embed_table, n=n, d=d)
    return out.reshape(input_ids.shape + (d,))
